# baseline (speedup 1.0000x reference)
import jax
import jax.numpy as jnp
from jax import lax
from jax.experimental import pallas as pl
from jax.experimental.pallas import tpu as pltpu

N = 524288
CH = 64
B = 16
GRID_H = 15
GRID_W = 20
CELLS = GRID_H * GRID_W
CELLS_PAD = 304
DECAY = 0.005
C_OUT = 9
C_PAD = 16

CHUNK_TC = 4096
TC_STEPS = N // CHUNK_TC
NEG = -1e30


def _tc_body(x_ref, aux_ref, ws_ref, bs_ref, wb_ref, bb_ref, wh_ref, bh_ref,
             grid_ref, acc, tacc):
    step = pl.program_id(0)

    @pl.when(step == 0)
    def _init():
        for b in range(B):
            tacc[b] = NEG
            acc[b] = jnp.zeros((CELLS_PAD, C_PAD), jnp.float32)

    xb = x_ref[...]
    h = jnp.maximum(
        jnp.dot(xb, ws_ref[...], preferred_element_type=jnp.float32) + bs_ref[...], 0.0)
    g = jnp.maximum(
        jnp.dot(h, wb_ref[...], preferred_element_type=jnp.float32) + bb_ref[...], 0.0)
    v = jnp.dot(g, wh_ref[...], preferred_element_type=jnp.float32) + bh_ref[...]

    px = aux_ref[:, 0:1]
    py = aux_ref[:, 1:2]
    ts = aux_ref[:, 2:3]
    bt = aux_ref[:, 3:4]

    lane8 = lax.broadcasted_iota(jnp.int32, (CHUNK_TC, 8), 1)
    adds = jnp.where(lane8 == 1, px, 0.0) + jnp.where(lane8 == 2, py, 0.0)
    va = v + adds
    m34 = (lane8 == 3) | (lane8 == 4)
    vb = jnp.where(m34, jnp.exp(jnp.minimum(va, 8.0)), va)
    wcol = jnp.exp(DECAY * ts)
    full = jnp.concatenate(
        [vb * wcol, wcol, jnp.zeros((CHUNK_TC, C_PAD - C_OUT), jnp.float32)],
        axis=1)

    gi = jnp.clip(va[:, 1:2].astype(jnp.int32), 0, GRID_W - 1)
    gj = jnp.clip(va[:, 2:3].astype(jnp.int32), 0, GRID_H - 1)
    gidx = bt.astype(jnp.int32) * CELLS_PAD + gj * GRID_W + gi
    full_bf = full.astype(jnp.bfloat16)

    b_first = aux_ref[0, 3]
    b_last = aux_ref[CHUNK_TC - 1, 3]
    bins = lax.broadcasted_iota(jnp.int32, (CHUNK_TC, CELLS_PAD), 1)
    for b in range(B):
        fb = jnp.float32(b)

        @pl.when((b_first <= fb) & (fb <= b_last))
        def _scatter(b=b):
            onehot = jnp.where(bins == gidx - b * CELLS_PAD,
                               1.0, 0.0).astype(jnp.bfloat16)
            contrib = lax.dot_general(
                onehot, full_bf, (((0,), (0,)), ((), ())),
                preferred_element_type=jnp.float32)
            acc[b] += contrib
            m = jnp.max(jnp.where(bt == jnp.float32(b), ts, NEG))
            tacc[b] = jnp.maximum(tacc[b], m)

    @pl.when(step == TC_STEPS - 1)
    def _fin():
        for b in range(B):
            sc = jnp.exp(-DECAY * jnp.maximum(tacc[b], 0.0))
            grid_ref[b] = acc[b] * sc


def _tc_forward(x, aux, ws, bs, wb, bb, wh, bh):
    return pl.pallas_call(
        _tc_body,
        grid=(TC_STEPS,),
        in_specs=[
            pl.BlockSpec((CHUNK_TC, CH), lambda i: (i, 0)),
            pl.BlockSpec((CHUNK_TC, 4), lambda i: (i, 0)),
            pl.BlockSpec((CH, CH), lambda i: (0, 0)),
            pl.BlockSpec((1, CH), lambda i: (0, 0)),
            pl.BlockSpec((CH, 3 * CH), lambda i: (0, 0)),
            pl.BlockSpec((1, 3 * CH), lambda i: (0, 0)),
            pl.BlockSpec((3 * CH, 8), lambda i: (0, 0)),
            pl.BlockSpec((1, 8), lambda i: (0, 0)),
        ],
        out_specs=pl.BlockSpec((B, CELLS_PAD, C_PAD), lambda i: (0, 0, 0)),
        out_shape=jax.ShapeDtypeStruct((B, CELLS_PAD, C_PAD), jnp.float32),
        scratch_shapes=[
            pltpu.VMEM((B, CELLS_PAD, C_PAD), jnp.float32),
            pltpu.SMEM((B,), jnp.float32),
        ],
        compiler_params=pltpu.CompilerParams(
            dimension_semantics=("arbitrary",)),
    )(x, aux, ws, bs, wb, bb, wh, bh)


def kernel(x, pos, batch, stem_w, stem_b, vote_conv_w, vote_conv_b,
           reg_conv_w, reg_conv_b, cls_conv_w, cls_conv_b,
           off_w, off_b, conf_w, conf_b, size_w, size_b, cls_w, cls_b):
    aux = jnp.concatenate([pos, batch.astype(jnp.float32)[:, None]], axis=1)
    wb = jnp.concatenate([vote_conv_w, reg_conv_w, cls_conv_w], axis=1)
    bb = jnp.concatenate([vote_conv_b, reg_conv_b, cls_conv_b])[None, :]
    wh = jnp.zeros((3 * CH, 8), jnp.float32)
    wh = wh.at[0:CH, 0:1].set(conf_w)
    wh = wh.at[0:CH, 1:3].set(off_w)
    wh = wh.at[CH:2 * CH, 3:5].set(size_w)
    wh = wh.at[2 * CH:3 * CH, 5:8].set(cls_w)
    bh = jnp.concatenate([conf_b, off_b, size_b, cls_b])[None, :]

    grid = _tc_forward(x, aux, stem_w, stem_b[None, :], wb, bb, wh, bh)
    g = grid[:, :CELLS, :C_OUT].reshape(B, GRID_H, GRID_W, C_OUT)
    return g.transpose(0, 3, 1, 2)


# speedup vs baseline: 3.4508x; 3.4508x over previous
import jax
import jax.numpy as jnp
from jax import lax
from jax.experimental import pallas as pl
from jax.experimental.pallas import tpu as pltpu

N = 524288
CH = 64
B = 16
GRID_H = 15
GRID_W = 20
CELLS = GRID_H * GRID_W
CELLS_PAD = 304
DECAY = 0.005
C_OUT = 9
C_PAD = 16

CHUNK_TC = 2048
TC_STEPS = N // CHUNK_TC
NEG = -1e30


def _tc_body(bf_ref, bl_ref, x_ref, aux_ref, ws_ref, bs_ref, wb_ref, bb_ref,
             wh_ref, bh_ref, grid_ref, acc, tacc):
    step = pl.program_id(0)

    @pl.when(step == 0)
    def _init():
        for b in range(B):
            tacc[b] = NEG
            acc[b] = jnp.zeros((CELLS_PAD, C_PAD), jnp.float32)

    xb = x_ref[...]
    h = jnp.maximum(
        jnp.dot(xb, ws_ref[...], preferred_element_type=jnp.float32) + bs_ref[...], 0.0)
    g = jnp.maximum(
        jnp.dot(h, wb_ref[...], preferred_element_type=jnp.float32) + bb_ref[...], 0.0)
    v = jnp.dot(g, wh_ref[...], preferred_element_type=jnp.float32) + bh_ref[...]

    px = aux_ref[:, 0:1]
    py = aux_ref[:, 1:2]
    ts = aux_ref[:, 2:3]
    bt = aux_ref[:, 3:4]

    lane8 = lax.broadcasted_iota(jnp.int32, (CHUNK_TC, 8), 1)
    adds = jnp.where(lane8 == 1, px, 0.0) + jnp.where(lane8 == 2, py, 0.0)
    va = v + adds
    m34 = (lane8 == 3) | (lane8 == 4)
    vb = jnp.where(m34, jnp.exp(jnp.minimum(va, 8.0)), va)
    wcol = jnp.exp(DECAY * ts)
    full = jnp.concatenate(
        [vb * wcol, wcol, jnp.zeros((CHUNK_TC, C_PAD - C_OUT), jnp.float32)],
        axis=1)

    gi = jnp.clip(va[:, 1:2].astype(jnp.int32), 0, GRID_W - 1)
    gj = jnp.clip(va[:, 2:3].astype(jnp.int32), 0, GRID_H - 1)
    gidx = bt.astype(jnp.int32) * CELLS_PAD + gj * GRID_W + gi

    step_i = pl.program_id(0)
    b_first = bf_ref[step_i]
    b_last = bl_ref[step_i]
    bins = lax.broadcasted_iota(jnp.int32, (CHUNK_TC, CELLS_PAD), 1)
    for b in range(B):
        @pl.when((b_first <= b) & (b <= b_last))
        def _scatter(b=b):
            onehot = jnp.where(bins == gidx - b * CELLS_PAD, 1.0, 0.0)
            contrib = lax.dot_general(
                onehot, full, (((0,), (0,)), ((), ())),
                preferred_element_type=jnp.float32)
            acc[b] += contrib
            m = jnp.max(jnp.where(bt == jnp.float32(b), ts, NEG))
            tacc[b] = jnp.maximum(tacc[b], m)

    @pl.when(step == TC_STEPS - 1)
    def _fin():
        for b in range(B):
            sc = jnp.exp(-DECAY * jnp.maximum(tacc[b], 0.0))
            grid_ref[b] = acc[b] * sc


def _tc_forward(bfirst, blast, x, aux, ws, bs, wb, bb, wh, bh):
    grid_spec = pltpu.PrefetchScalarGridSpec(
        num_scalar_prefetch=2,
        grid=(TC_STEPS,),
        in_specs=[
            pl.BlockSpec((CHUNK_TC, CH), lambda i, bf, bl: (i, 0)),
            pl.BlockSpec((CHUNK_TC, 4), lambda i, bf, bl: (i, 0)),
            pl.BlockSpec((CH, CH), lambda i, bf, bl: (0, 0)),
            pl.BlockSpec((1, CH), lambda i, bf, bl: (0, 0)),
            pl.BlockSpec((CH, 3 * CH), lambda i, bf, bl: (0, 0)),
            pl.BlockSpec((1, 3 * CH), lambda i, bf, bl: (0, 0)),
            pl.BlockSpec((3 * CH, 8), lambda i, bf, bl: (0, 0)),
            pl.BlockSpec((1, 8), lambda i, bf, bl: (0, 0)),
        ],
        out_specs=pl.BlockSpec((B, CELLS_PAD, C_PAD), lambda i, bf, bl: (0, 0, 0)),
        scratch_shapes=[
            pltpu.VMEM((B, CELLS_PAD, C_PAD), jnp.float32),
            pltpu.SMEM((B,), jnp.float32),
        ],
    )
    return pl.pallas_call(
        _tc_body,
        grid_spec=grid_spec,
        out_shape=jax.ShapeDtypeStruct((B, CELLS_PAD, C_PAD), jnp.float32),
        compiler_params=pltpu.CompilerParams(
            dimension_semantics=("arbitrary",)),
    )(bfirst, blast, x, aux, ws, bs, wb, bb, wh, bh)


def kernel(x, pos, batch, stem_w, stem_b, vote_conv_w, vote_conv_b,
           reg_conv_w, reg_conv_b, cls_conv_w, cls_conv_b,
           off_w, off_b, conf_w, conf_b, size_w, size_b, cls_w, cls_b):
    aux = jnp.concatenate([pos, batch.astype(jnp.float32)[:, None]], axis=1)
    wb = jnp.concatenate([vote_conv_w, reg_conv_w, cls_conv_w], axis=1)
    bb = jnp.concatenate([vote_conv_b, reg_conv_b, cls_conv_b])[None, :]
    wh = jnp.zeros((3 * CH, 8), jnp.float32)
    wh = wh.at[0:CH, 0:1].set(conf_w)
    wh = wh.at[0:CH, 1:3].set(off_w)
    wh = wh.at[CH:2 * CH, 3:5].set(size_w)
    wh = wh.at[2 * CH:3 * CH, 5:8].set(cls_w)
    bh = jnp.concatenate([conf_b, off_b, size_b, cls_b])[None, :]

    bfirst = batch[::CHUNK_TC]
    blast = batch[CHUNK_TC - 1::CHUNK_TC]
    grid = _tc_forward(bfirst, blast, x, aux, stem_w, stem_b[None, :],
                       wb, bb, wh, bh)
    g = grid[:, :CELLS, :C_OUT].reshape(B, GRID_H, GRID_W, C_OUT)
    return g.transpose(0, 3, 1, 2)


# speedup vs baseline: 3.6493x; 1.0575x over previous
import jax
import jax.numpy as jnp
from jax import lax
from jax.experimental import pallas as pl
from jax.experimental.pallas import tpu as pltpu

N = 524288
CH = 64
B = 16
GRID_H = 15
GRID_W = 20
CELLS = GRID_H * GRID_W
CELLS_PAD = 304
DECAY = 0.005
C_OUT = 9
C_PAD = 16

CHUNK_TC = 2048
TC_STEPS = N // CHUNK_TC
NEG = -1e30


def _tc_body(bf_ref, bl_ref, x_ref, aux_ref, ws_ref, bs_ref, wb_ref, bb_ref,
             wh_ref, bh_ref, grid_ref, acc, tacc):
    step = pl.program_id(0)

    @pl.when(step == 0)
    def _init():
        for b in range(B):
            tacc[b] = NEG
            acc[b] = jnp.zeros((C_PAD, CELLS_PAD), jnp.float32)

    xb = x_ref[...]
    h = jnp.maximum(
        jnp.dot(xb, ws_ref[...], preferred_element_type=jnp.float32) + bs_ref[...], 0.0)
    g = jnp.maximum(
        jnp.dot(h, wb_ref[...], preferred_element_type=jnp.float32) + bb_ref[...], 0.0)
    v = jnp.dot(g, wh_ref[...], preferred_element_type=jnp.float32) + bh_ref[...]

    px = aux_ref[:, 0:1]
    py = aux_ref[:, 1:2]
    ts = aux_ref[:, 2:3]
    bt = aux_ref[:, 3:4]

    lane8 = lax.broadcasted_iota(jnp.int32, (CHUNK_TC, 8), 1)
    adds = jnp.where(lane8 == 1, px, 0.0) + jnp.where(lane8 == 2, py, 0.0)
    va = v + adds
    m34 = (lane8 == 3) | (lane8 == 4)
    vb = jnp.where(m34, jnp.exp(jnp.minimum(va, 8.0)), va)
    wcol = jnp.exp(DECAY * ts)
    full = jnp.concatenate(
        [vb * wcol, wcol, jnp.zeros((CHUNK_TC, C_PAD - C_OUT), jnp.float32)],
        axis=1)

    gi = jnp.clip(va[:, 1:2].astype(jnp.int32), 0, GRID_W - 1)
    gj = jnp.clip(va[:, 2:3].astype(jnp.int32), 0, GRID_H - 1)
    gidx = bt.astype(jnp.int32) * CELLS_PAD + gj * GRID_W + gi

    step_i = pl.program_id(0)
    b_first = bf_ref[step_i]
    b_last = bl_ref[step_i]
    bins = lax.broadcasted_iota(jnp.int32, (CHUNK_TC, CELLS_PAD), 1)
    for b in range(B):
        @pl.when((b_first <= b) & (b <= b_last))
        def _scatter(b=b):
            onehot = jnp.where(bins == gidx - b * CELLS_PAD, 1.0, 0.0)
            contrib = lax.dot_general(
                full, onehot, (((0,), (0,)), ((), ())),
                preferred_element_type=jnp.float32)
            acc[b] += contrib
            m = jnp.max(jnp.where(bt == jnp.float32(b), ts, NEG))
            tacc[b] = jnp.maximum(tacc[b], m)

    @pl.when(step == TC_STEPS - 1)
    def _fin():
        for b in range(B):
            sc = jnp.exp(-DECAY * jnp.maximum(tacc[b], 0.0))
            grid_ref[b] = acc[b] * sc


def _tc_forward(bfirst, blast, x, aux, ws, bs, wb, bb, wh, bh):
    grid_spec = pltpu.PrefetchScalarGridSpec(
        num_scalar_prefetch=2,
        grid=(TC_STEPS,),
        in_specs=[
            pl.BlockSpec((CHUNK_TC, CH), lambda i, bf, bl: (i, 0)),
            pl.BlockSpec((CHUNK_TC, 4), lambda i, bf, bl: (i, 0)),
            pl.BlockSpec((CH, CH), lambda i, bf, bl: (0, 0)),
            pl.BlockSpec((1, CH), lambda i, bf, bl: (0, 0)),
            pl.BlockSpec((CH, 3 * CH), lambda i, bf, bl: (0, 0)),
            pl.BlockSpec((1, 3 * CH), lambda i, bf, bl: (0, 0)),
            pl.BlockSpec((3 * CH, 8), lambda i, bf, bl: (0, 0)),
            pl.BlockSpec((1, 8), lambda i, bf, bl: (0, 0)),
        ],
        out_specs=pl.BlockSpec((B, C_PAD, CELLS_PAD), lambda i, bf, bl: (0, 0, 0)),
        scratch_shapes=[
            pltpu.VMEM((B, C_PAD, CELLS_PAD), jnp.float32),
            pltpu.SMEM((B,), jnp.float32),
        ],
    )
    return pl.pallas_call(
        _tc_body,
        grid_spec=grid_spec,
        out_shape=jax.ShapeDtypeStruct((B, C_PAD, CELLS_PAD), jnp.float32),
        compiler_params=pltpu.CompilerParams(
            dimension_semantics=("arbitrary",)),
    )(bfirst, blast, x, aux, ws, bs, wb, bb, wh, bh)


def kernel(x, pos, batch, stem_w, stem_b, vote_conv_w, vote_conv_b,
           reg_conv_w, reg_conv_b, cls_conv_w, cls_conv_b,
           off_w, off_b, conf_w, conf_b, size_w, size_b, cls_w, cls_b):
    aux = jnp.concatenate([pos, batch.astype(jnp.float32)[:, None]], axis=1)
    wb = jnp.concatenate([vote_conv_w, reg_conv_w, cls_conv_w], axis=1)
    bb = jnp.concatenate([vote_conv_b, reg_conv_b, cls_conv_b])[None, :]
    wh = jnp.zeros((3 * CH, 8), jnp.float32)
    wh = wh.at[0:CH, 0:1].set(conf_w)
    wh = wh.at[0:CH, 1:3].set(off_w)
    wh = wh.at[CH:2 * CH, 3:5].set(size_w)
    wh = wh.at[2 * CH:3 * CH, 5:8].set(cls_w)
    bh = jnp.concatenate([conf_b, off_b, size_b, cls_b])[None, :]

    bfirst = batch[::CHUNK_TC]
    blast = batch[CHUNK_TC - 1::CHUNK_TC]
    grid = _tc_forward(bfirst, blast, x, aux, stem_w, stem_b[None, :],
                       wb, bb, wh, bh)
    return grid[:, :C_OUT, :CELLS].reshape(B, C_OUT, GRID_H, GRID_W)


# speedup vs baseline: 3.7921x; 1.0391x over previous
import jax
import jax.numpy as jnp
from jax import lax
from jax.experimental import pallas as pl
from jax.experimental.pallas import tpu as pltpu

N = 524288
CH = 64
B = 16
GRID_H = 15
GRID_W = 20
CELLS = GRID_H * GRID_W
CELLS_PAD = 304
DECAY = 0.005
C_OUT = 9
C_PAD = 16

CHUNK_TC = 4096
TC_STEPS = N // CHUNK_TC
NEG = -1e30


def _tc_body(bf_ref, bl_ref, x_ref, aux_ref, ws_ref, bs_ref, wb_ref, bb_ref,
             wh_ref, bh_ref, grid_ref, acc, tacc):
    step = pl.program_id(0)

    @pl.when(step == 0)
    def _init():
        for b in range(B):
            tacc[b] = NEG
            acc[b] = jnp.zeros((C_PAD, CELLS_PAD), jnp.float32)

    xb = x_ref[...]
    h = jnp.maximum(
        jnp.dot(xb, ws_ref[...], preferred_element_type=jnp.float32) + bs_ref[...], 0.0)
    g = jnp.maximum(
        jnp.dot(h, wb_ref[...], preferred_element_type=jnp.float32) + bb_ref[...], 0.0)
    v = jnp.dot(g, wh_ref[...], preferred_element_type=jnp.float32) + bh_ref[...]

    px = aux_ref[:, 0:1]
    py = aux_ref[:, 1:2]
    ts = aux_ref[:, 2:3]
    bt = aux_ref[:, 3:4]

    lane8 = lax.broadcasted_iota(jnp.int32, (CHUNK_TC, 8), 1)
    adds = jnp.where(lane8 == 1, px, 0.0) + jnp.where(lane8 == 2, py, 0.0)
    va = v + adds
    m34 = (lane8 == 3) | (lane8 == 4)
    vb = jnp.where(m34, jnp.exp(jnp.minimum(va, 8.0)), va)
    wcol = jnp.exp(DECAY * ts)
    full = jnp.concatenate(
        [vb * wcol, wcol, jnp.zeros((CHUNK_TC, C_PAD - C_OUT), jnp.float32)],
        axis=1)

    gi = jnp.clip(va[:, 1:2].astype(jnp.int32), 0, GRID_W - 1)
    gj = jnp.clip(va[:, 2:3].astype(jnp.int32), 0, GRID_H - 1)
    gidx = bt.astype(jnp.int32) * CELLS_PAD + gj * GRID_W + gi

    step_i = pl.program_id(0)
    b_first = bf_ref[step_i]
    b_last = bl_ref[step_i]
    bins = lax.broadcasted_iota(jnp.int32, (CHUNK_TC, CELLS_PAD), 1)
    for b in range(B):
        @pl.when((b_first <= b) & (b <= b_last))
        def _scatter(b=b):
            onehot = jnp.where(bins == gidx - b * CELLS_PAD, 1.0, 0.0)
            contrib = lax.dot_general(
                full, onehot, (((0,), (0,)), ((), ())),
                preferred_element_type=jnp.float32)
            acc[b] += contrib
            m = jnp.max(jnp.where(bt == jnp.float32(b), ts, NEG))
            tacc[b] = jnp.maximum(tacc[b], m)

    @pl.when(step == TC_STEPS - 1)
    def _fin():
        for b in range(B):
            sc = jnp.exp(-DECAY * jnp.maximum(tacc[b], 0.0))
            grid_ref[b] = acc[b] * sc


def _tc_forward(bfirst, blast, x, aux, ws, bs, wb, bb, wh, bh):
    grid_spec = pltpu.PrefetchScalarGridSpec(
        num_scalar_prefetch=2,
        grid=(TC_STEPS,),
        in_specs=[
            pl.BlockSpec((CHUNK_TC, CH), lambda i, bf, bl: (i, 0)),
            pl.BlockSpec((CHUNK_TC, 4), lambda i, bf, bl: (i, 0)),
            pl.BlockSpec((CH, CH), lambda i, bf, bl: (0, 0)),
            pl.BlockSpec((1, CH), lambda i, bf, bl: (0, 0)),
            pl.BlockSpec((CH, 3 * CH), lambda i, bf, bl: (0, 0)),
            pl.BlockSpec((1, 3 * CH), lambda i, bf, bl: (0, 0)),
            pl.BlockSpec((3 * CH, 8), lambda i, bf, bl: (0, 0)),
            pl.BlockSpec((1, 8), lambda i, bf, bl: (0, 0)),
        ],
        out_specs=pl.BlockSpec((B, C_PAD, CELLS_PAD), lambda i, bf, bl: (0, 0, 0)),
        scratch_shapes=[
            pltpu.VMEM((B, C_PAD, CELLS_PAD), jnp.float32),
            pltpu.SMEM((B,), jnp.float32),
        ],
    )
    return pl.pallas_call(
        _tc_body,
        grid_spec=grid_spec,
        out_shape=jax.ShapeDtypeStruct((B, C_PAD, CELLS_PAD), jnp.float32),
        compiler_params=pltpu.CompilerParams(
            dimension_semantics=("arbitrary",)),
    )(bfirst, blast, x, aux, ws, bs, wb, bb, wh, bh)


def kernel(x, pos, batch, stem_w, stem_b, vote_conv_w, vote_conv_b,
           reg_conv_w, reg_conv_b, cls_conv_w, cls_conv_b,
           off_w, off_b, conf_w, conf_b, size_w, size_b, cls_w, cls_b):
    aux = jnp.concatenate([pos, batch.astype(jnp.float32)[:, None]], axis=1)
    wb = jnp.concatenate([vote_conv_w, reg_conv_w, cls_conv_w], axis=1)
    bb = jnp.concatenate([vote_conv_b, reg_conv_b, cls_conv_b])[None, :]
    wh = jnp.zeros((3 * CH, 8), jnp.float32)
    wh = wh.at[0:CH, 0:1].set(conf_w)
    wh = wh.at[0:CH, 1:3].set(off_w)
    wh = wh.at[CH:2 * CH, 3:5].set(size_w)
    wh = wh.at[2 * CH:3 * CH, 5:8].set(cls_w)
    bh = jnp.concatenate([conf_b, off_b, size_b, cls_b])[None, :]

    bfirst = batch[::CHUNK_TC]
    blast = batch[CHUNK_TC - 1::CHUNK_TC]
    grid = _tc_forward(bfirst, blast, x, aux, stem_w, stem_b[None, :],
                       wb, bb, wh, bh)
    return grid[:, :C_OUT, :CELLS].reshape(B, C_OUT, GRID_H, GRID_W)
